# initial kernel scaffold (unmeasured)
import jax
import jax.numpy as jnp
from jax import lax
from jax.experimental import pallas as pl
from jax.experimental.pallas import tpu as pltpu

_sem_signal = getattr(pl, "semaphore_signal", None) or pltpu.semaphore_signal
_sem_wait = getattr(pl, "semaphore_wait", None) or pltpu.semaphore_wait
_CompilerParams = getattr(pltpu, "CompilerParams", None) or pltpu.TPUCompilerParams


def kernel(Q, K, V):
    b, q, h, d = Q.shape
    _, kl, _, _ = K.shape
    scale = d ** -0.5

    def body(q_ref, k_ref, v_ref, out_ref, acc_ref, recv_ref, send_sems, recv_sems):
        bi = pl.program_id(0)
        nb = pl.num_programs(0)

        for hi in range(h):
            qh = q_ref[0, :, hi, :]
            kh = k_ref[0, :, hi, :]
            s = lax.dot_general(
                qh, kh, (((1,), (1,)), ((), ())),
                preferred_element_type=jnp.float32,
            ) * scale
            p = jnp.exp(s)
            l = jnp.sum(p, axis=1, keepdims=True)
            vh = v_ref[0, :, hi, :]
            o = lax.dot_general(
                p, vh, (((1,), (0,)), ((), ())),
                preferred_element_type=jnp.float32,
            )
            acc_ref[bi, :, hi, :] = o
            acc_ref[bi, :, h, hi:hi + 1] = l

        @pl.when(bi == nb - 1)
        def _():
            my_x = lax.axis_index("x")
            my_y = lax.axis_index("y")
            my_z = lax.axis_index("z")

            barrier = pltpu.get_barrier_semaphore()
            for dist in (1, 2):
                _sem_signal(
                    barrier, inc=1,
                    device_id=(my_x, my_y ^ dist, my_z),
                    device_id_type=pl.DeviceIdType.MESH,
                )
            _sem_wait(barrier, 2)

            for step, dist in enumerate((1, 2)):
                rdma = pltpu.make_async_remote_copy(
                    src_ref=acc_ref,
                    dst_ref=recv_ref,
                    send_sem=send_sems.at[step],
                    recv_sem=recv_sems.at[step],
                    device_id=(my_x, my_y ^ dist, my_z),
                    device_id_type=pl.DeviceIdType.MESH,
                )
                rdma.start()
                rdma.wait()
                acc_ref[:, :, :, :] = acc_ref[:, :, :, :] + recv_ref[:, :, :, :]

            for bb in range(b):
                for hi in range(h):
                    out_ref[bb, :, hi, :] = (
                        acc_ref[bb, :, hi, :] / acc_ref[bb, :, h, hi:hi + 1]
                    )

    return pl.pallas_call(
        body,
        grid=(b,),
        out_shape=jax.ShapeDtypeStruct((b, q, h, d), jnp.float32),
        in_specs=[
            pl.BlockSpec((1, q, h, d), lambda i: (i, 0, 0, 0)),
            pl.BlockSpec((1, kl, h, d), lambda i: (i, 0, 0, 0)),
            pl.BlockSpec((1, kl, h, d), lambda i: (i, 0, 0, 0)),
        ],
        out_specs=pl.BlockSpec((b, q, h, d), lambda i: (0, 0, 0, 0)),
        scratch_shapes=[
            pltpu.VMEM((b, q, h + 1, d), jnp.float32),
            pltpu.VMEM((b, q, h + 1, d), jnp.float32),
            pltpu.SemaphoreType.DMA((2,)),
            pltpu.SemaphoreType.DMA((2,)),
        ],
        compiler_params=_CompilerParams(
            collective_id=0,
            dimension_semantics=("arbitrary",),
        ),
    )(Q, K, V)


# baseline (device time: 134630 ns/iter reference)
import jax

jax.config.update("jax_compilation_cache_dir", "/tmp/jax_pallas_cache")
jax.config.update("jax_persistent_cache_min_compile_time_secs", 0)

import jax.numpy as jnp

try:
    jax.block_until_ready(jnp.zeros((8, 128), jnp.float32) + 1.0)
except Exception:
    pass
from jax import lax
from jax.experimental import pallas as pl
from jax.experimental.pallas import tpu as pltpu

_sem_signal = getattr(pl, "semaphore_signal", None) or pltpu.semaphore_signal
_sem_wait = getattr(pl, "semaphore_wait", None) or pltpu.semaphore_wait
_CompilerParams = getattr(pltpu, "CompilerParams", None) or pltpu.TPUCompilerParams


def kernel(Q, K, V):
    b, q, h, d = Q.shape
    _, kl, _, _ = K.shape
    scale = d ** -0.5

    nkc = 2
    kc = kl // nkc

    def body(q_ref, k_ref, v_ref, out_ref, acc_ref, recv_ref, send_sems, recv_sems):
        bi = pl.program_id(0)
        nb = pl.num_programs(0)
        ci = pl.program_id(1)
        nc = pl.num_programs(1)

        for hi in range(h):
            qh = q_ref[0, :, hi, :]
            kh = k_ref[0, :, hi, :]
            s = lax.dot_general(
                qh, kh, (((1,), (1,)), ((), ())),
                preferred_element_type=jnp.float32,
            ) * scale
            p = jnp.exp(s)
            l = jnp.sum(p, axis=1, keepdims=True)
            vh = v_ref[0, :, hi, :]
            o = lax.dot_general(
                p, vh, (((1,), (0,)), ((), ())),
                preferred_element_type=jnp.float32,
            )

            @pl.when(ci == 0)
            def _():
                acc_ref[bi, :, hi, :] = o
                acc_ref[bi, :, h, hi:hi + 1] = l

            @pl.when(ci != 0)
            def _():
                acc_ref[bi, :, hi, :] = acc_ref[bi, :, hi, :] + o
                acc_ref[bi, :, h, hi:hi + 1] = acc_ref[bi, :, h, hi:hi + 1] + l

        @pl.when((bi == nb - 1) & (ci == nc - 1))
        def _():
            my_x = lax.axis_index("x")
            my_y = lax.axis_index("y")
            my_z = lax.axis_index("z")

            barrier = pltpu.get_barrier_semaphore()
            for dist in (1, 2):
                _sem_signal(
                    barrier, inc=1,
                    device_id=(my_x, my_y ^ dist, my_z),
                    device_id_type=pl.DeviceIdType.MESH,
                )
            _sem_wait(barrier, 2)

            for step, dist in enumerate((1, 2)):
                rdma = pltpu.make_async_remote_copy(
                    src_ref=acc_ref,
                    dst_ref=recv_ref.at[step],
                    send_sem=send_sems.at[step],
                    recv_sem=recv_sems.at[step],
                    device_id=(my_x, my_y ^ dist, my_z),
                    device_id_type=pl.DeviceIdType.MESH,
                )
                rdma.start()
                rdma.wait()
                acc_ref[:, :, :, :] = acc_ref[:, :, :, :] + recv_ref[step, :, :, :, :]

            for hi in range(h):
                out_ref[:, :, hi, :] = (
                    acc_ref[:, :, hi, :] / acc_ref[:, :, h, hi:hi + 1]
                )

    return pl.pallas_call(
        body,
        grid=(b, nkc),
        out_shape=jax.ShapeDtypeStruct((b, q, h, d), jnp.float32),
        in_specs=[
            pl.BlockSpec((1, q, h, d), lambda i, j: (i, 0, 0, 0)),
            pl.BlockSpec((1, kc, h, d), lambda i, j: (i, j, 0, 0)),
            pl.BlockSpec((1, kc, h, d), lambda i, j: (i, j, 0, 0)),
        ],
        out_specs=pl.BlockSpec((b, q, h, d), lambda i, j: (0, 0, 0, 0)),
        scratch_shapes=[
            pltpu.VMEM((b, q, h + 1, d), jnp.float32),
            pltpu.VMEM((2, b, q, h + 1, d), jnp.float32),
            pltpu.SemaphoreType.DMA((2,)),
            pltpu.SemaphoreType.DMA((2,)),
        ],
        compiler_params=_CompilerParams(
            collective_id=0,
            dimension_semantics=("arbitrary", "arbitrary"),
        ),
    )(Q, K, V)
